# Initial kernel scaffold; baseline (speedup 1.0000x reference)
import functools

import jax
import jax.numpy as jnp
from jax import lax
from jax.experimental import pallas as pl
from jax.experimental.pallas import tpu as pltpu
from jax.experimental.pallas import tpu_sc as plsc

N_NODES = 10000
DIM = 256
HALF = 128
N_SEG = 10000
N_ENTRIES = 160000

NC = 2
NS = 16
L = 16

CHUNK = 128
SEG_PAD = 10240
E_PAD = 161792
E_PER_SUB = E_PAD // NS
N_CHUNKS = E_PER_SUB // CHUNK
ZROWS = SEG_PAD // NS
SEG_PER_SUB = N_SEG // NS
FIN_CHUNK = 125

_mesh = plsc.VectorSubcoreMesh(core_axis_name="c", subcore_axis_name="s")


@jax.jit
def _sc_aggregate(table2, nid, sid):
    @functools.partial(
        pl.kernel,
        out_type=jax.ShapeDtypeStruct((NC, N_SEG, HALF), jnp.float32),
        mesh=_mesh,
        scratch_types=[
            pltpu.VMEM((1, CHUNK), jnp.int32),
            pltpu.VMEM((1, CHUNK), jnp.int32),
            pltpu.VMEM((CHUNK, HALF), jnp.float32),
            pltpu.VMEM((CHUNK, L), jnp.float32),
            pltpu.VMEM((CHUNK, L), jnp.float32),
            pltpu.VMEM_SHARED((SEG_PAD, HALF), jnp.float32),
            pltpu.VMEM_SHARED((SEG_PAD, L), jnp.float32),
        ],
    )
    def k(table_hbm, nid_hbm, sid_hbm, out_hbm,
          nid_v, sid_v, rows_v, ones_v, cnt_v, acc_sh, cnt_sh):
        cid = lax.axis_index("c")
        sub = lax.axis_index("s")
        row_off = cid * N_NODES

        zero16 = jnp.zeros((L,), jnp.float32)
        one16 = jnp.ones((L,), jnp.float32)

        @pl.loop(0, CHUNK)
        def _(i):
            for j in range(HALF // L):
                rows_v[i, pl.ds(j * L, L)] = zero16
            ones_v[i, pl.ds(0, L)] = one16
            cnt_v[i, pl.ds(0, L)] = zero16

        @pl.loop(0, ZROWS // CHUNK)
        def _(z):
            zbase = sub * ZROWS + z * CHUNK
            pltpu.sync_copy(rows_v, acc_sh.at[pl.ds(zbase, CHUNK)])
            pltpu.sync_copy(cnt_v, cnt_sh.at[pl.ds(zbase, CHUNK)])

        plsc.subcore_barrier()

        @pl.loop(0, N_CHUNKS)
        def _(c):
            e0 = sub * E_PER_SUB + c * CHUNK
            pltpu.sync_copy(nid_hbm.at[pl.ds(e0, CHUNK)], nid_v.at[0])
            pltpu.sync_copy(sid_hbm.at[pl.ds(e0, CHUNK)], sid_v.at[0])
            for j in range(CHUNK // L):
                nid_v[0, pl.ds(j * L, L)] = nid_v[0, pl.ds(j * L, L)] + row_off
            pltpu.sync_copy(table_hbm.at[nid_v.at[0]], rows_v)
            pltpu.sync_copy(rows_v, acc_sh.at[sid_v.at[0]], add=True)
            pltpu.sync_copy(ones_v, cnt_sh.at[sid_v.at[0]], add=True)

        plsc.subcore_barrier()

        @pl.loop(0, SEG_PER_SUB // FIN_CHUNK)
        def _(f):
            base = sub * SEG_PER_SUB + f * FIN_CHUNK
            pltpu.sync_copy(acc_sh.at[pl.ds(base, FIN_CHUNK)],
                            rows_v.at[pl.ds(0, FIN_CHUNK)])
            pltpu.sync_copy(cnt_sh.at[pl.ds(base, FIN_CHUNK)],
                            cnt_v.at[pl.ds(0, FIN_CHUNK)])

            @pl.loop(0, FIN_CHUNK)
            def _(i):
                cnt = cnt_v[i, pl.ds(0, L)]
                inv = 1.0 / jnp.maximum(cnt, 1.0)
                for j in range(HALF // L):
                    rows_v[i, pl.ds(j * L, L)] = (
                        rows_v[i, pl.ds(j * L, L)] * inv)

            pltpu.sync_copy(rows_v.at[pl.ds(0, FIN_CHUNK)],
                            out_hbm.at[cid, pl.ds(base, FIN_CHUNK)])

    return k(table2, nid, sid)


def kernel(embedding_table, node_ids, segment_ids):
    nid = node_ids.astype(jnp.int32)
    sid = segment_ids.astype(jnp.int32)
    pad = E_PAD - N_ENTRIES
    nid = jnp.concatenate([nid, jnp.zeros((pad,), jnp.int32)])
    sid = jnp.concatenate([sid, jnp.full((pad,), N_SEG, jnp.int32)])
    table2 = jnp.concatenate(
        [embedding_table[:, :HALF], embedding_table[:, HALF:]], axis=0)
    halves = _sc_aggregate(table2, nid, sid)
    return jnp.concatenate([halves[0], halves[1]], axis=1)


# speedup vs baseline: 2.5785x; 2.5785x over previous
import functools

import jax
import jax.numpy as jnp
from jax import lax
from jax.experimental import pallas as pl
from jax.experimental.pallas import tpu as pltpu
from jax.experimental.pallas import tpu_sc as plsc

N_NODES = 10000
DIM = 256
QUART = 64
N_SEG = 10000
N_ENTRIES = 160000

NC = 2
NS = 16
L = 16

CHUNK = 128
SEG_PAD = 10112
SEG_CHUNKS = SEG_PAD // CHUNK
E_PAD = 161792
E_PER_SUB = E_PAD // NS
N_CHUNKS = E_PER_SUB // CHUNK

_mesh = plsc.VectorSubcoreMesh(core_axis_name="c", subcore_axis_name="s")


@jax.jit
def _sc_aggregate(table4, nid, sid):
    @functools.partial(
        pl.kernel,
        out_type=jax.ShapeDtypeStruct((2 * NC, SEG_PAD, QUART), jnp.float32),
        mesh=_mesh,
        compiler_params=pltpu.CompilerParams(use_tc_tiling_on_sc=False),
        scratch_types=[
            pltpu.VMEM((1, CHUNK), jnp.int32),
            pltpu.VMEM((1, CHUNK), jnp.int32),
            pltpu.VMEM((CHUNK, QUART), jnp.float32),
            pltpu.VMEM((CHUNK, QUART), jnp.float32),
            pltpu.VMEM((CHUNK, L), jnp.float32),
            pltpu.VMEM((CHUNK, L), jnp.float32),
            pltpu.VMEM_SHARED((SEG_PAD, QUART), jnp.float32),
            pltpu.VMEM_SHARED((SEG_PAD, L), jnp.float32),
        ],
    )
    def k(table_hbm, nid_hbm, sid_hbm, out_hbm,
          nid_v, sid_v, rows_v, zero_v, ones_v, cnt_v, acc_sh, cnt_sh):
        cid = lax.axis_index("c")
        sub = lax.axis_index("s")

        zero16 = jnp.zeros((L,), jnp.float32)
        one16 = jnp.ones((L,), jnp.float32)

        @pl.loop(0, CHUNK)
        def _(i):
            for j in range(QUART // L):
                zero_v[i, pl.ds(j * L, L)] = zero16
            ones_v[i, pl.ds(0, L)] = one16
            cnt_v[i, pl.ds(0, L)] = zero16

        @pl.loop(0, pl.cdiv(SEG_CHUNKS, NS))
        def _(z):
            t = z * NS + sub

            @pl.when(t < SEG_CHUNKS)
            def _():
                pltpu.sync_copy(zero_v, acc_sh.at[pl.ds(t * CHUNK, CHUNK)])
                pltpu.sync_copy(cnt_v, cnt_sh.at[pl.ds(t * CHUNK, CHUNK)])

        plsc.subcore_barrier()

        def accumulate(quart, with_counts):
            row_off = quart * N_NODES

            @pl.loop(0, N_CHUNKS)
            def _(c):
                e0 = sub * E_PER_SUB + c * CHUNK
                pltpu.sync_copy(nid_hbm.at[pl.ds(e0, CHUNK)], nid_v.at[0])
                pltpu.sync_copy(sid_hbm.at[pl.ds(e0, CHUNK)], sid_v.at[0])
                for j in range(CHUNK // L):
                    nid_v[0, pl.ds(j * L, L)] = (
                        nid_v[0, pl.ds(j * L, L)] + row_off)
                pltpu.sync_copy(table_hbm.at[nid_v.at[0]], rows_v)
                pltpu.sync_copy(rows_v, acc_sh.at[sid_v.at[0]], add=True)
                if with_counts:
                    pltpu.sync_copy(ones_v, cnt_sh.at[sid_v.at[0]], add=True)

        def finalize(quart, rezero):
            @pl.loop(0, pl.cdiv(SEG_CHUNKS, NS))
            def _(f):
                t = f * NS + sub

                @pl.when(t < SEG_CHUNKS)
                def _():
                    base = t * CHUNK
                    pltpu.sync_copy(acc_sh.at[pl.ds(base, CHUNK)], rows_v)
                    pltpu.sync_copy(cnt_sh.at[pl.ds(base, CHUNK)], cnt_v)
                    if rezero:
                        pltpu.sync_copy(zero_v,
                                        acc_sh.at[pl.ds(base, CHUNK)])

                    @pl.loop(0, CHUNK)
                    def _(i):
                        cnt = cnt_v[i, pl.ds(0, L)]
                        inv = 1.0 / jnp.maximum(cnt, 1.0)
                        for j in range(QUART // L):
                            rows_v[i, pl.ds(j * L, L)] = (
                                rows_v[i, pl.ds(j * L, L)] * inv)

                    pltpu.sync_copy(rows_v,
                                    out_hbm.at[quart, pl.ds(base, CHUNK)])

        accumulate(2 * cid, True)
        plsc.subcore_barrier()
        finalize(2 * cid, True)
        plsc.subcore_barrier()
        accumulate(2 * cid + 1, False)
        plsc.subcore_barrier()
        finalize(2 * cid + 1, False)

    return k(table4, nid, sid)


def kernel(embedding_table, node_ids, segment_ids):
    nid = node_ids.astype(jnp.int32)
    sid = segment_ids.astype(jnp.int32)
    pad = E_PAD - N_ENTRIES
    nid = jnp.concatenate([nid, jnp.zeros((pad,), jnp.int32)])
    sid = jnp.concatenate([sid, jnp.full((pad,), N_SEG, jnp.int32)])
    table4 = jnp.concatenate(
        [embedding_table[:, q * QUART:(q + 1) * QUART] for q in range(4)],
        axis=0)
    quarters = _sc_aggregate(table4, nid, sid)
    return jnp.concatenate([quarters[q, :N_SEG] for q in range(4)], axis=1)


# speedup vs baseline: 3.3302x; 1.2916x over previous
import functools

import jax
import jax.numpy as jnp
from jax import lax
from jax.experimental import pallas as pl
from jax.experimental.pallas import tpu as pltpu
from jax.experimental.pallas import tpu_sc as plsc

N_NODES = 10000
DIM = 256
QUART = 64
N_SEG = 10000
N_ENTRIES = 160000

NC = 2
NS = 16
L = 16

CHUNK = 128
SEG_PAD = 10112
SEG_CHUNKS = SEG_PAD // CHUNK
N_CHUNKS = 80
E_PER_SUB = N_CHUNKS * CHUNK
E_PAD = NS * E_PER_SUB

_mesh = plsc.VectorSubcoreMesh(core_axis_name="c", subcore_axis_name="s")


@jax.jit
def _sc_aggregate(table4, nid, sid):
    @functools.partial(
        pl.kernel,
        out_type=jax.ShapeDtypeStruct((2 * NC, SEG_PAD, QUART), jnp.float32),
        mesh=_mesh,
        compiler_params=pltpu.CompilerParams(use_tc_tiling_on_sc=False),
        scratch_types=[
            pltpu.VMEM((N_CHUNKS, CHUNK), jnp.int32),
            pltpu.VMEM((N_CHUNKS, CHUNK), jnp.int32),
            pltpu.VMEM((CHUNK, QUART), jnp.float32),
            pltpu.VMEM((CHUNK, QUART), jnp.float32),
            pltpu.VMEM((CHUNK, QUART), jnp.float32),
            pltpu.VMEM((CHUNK, L), jnp.float32),
            pltpu.VMEM((CHUNK, L), jnp.float32),
            pltpu.VMEM_SHARED((SEG_PAD, QUART), jnp.float32),
            pltpu.VMEM_SHARED((SEG_PAD, L), jnp.float32),
            pltpu.SemaphoreType.DMA,
            pltpu.SemaphoreType.DMA,
            pltpu.SemaphoreType.DMA,
            pltpu.SemaphoreType.DMA,
        ],
    )
    def k(table_hbm, nid_hbm, sid_hbm, out_hbm,
          nid_v, sid_v, rows0, rows1, zero_v, ones_v, cnt_v, acc_sh, cnt_sh,
          gsem0, gsem1, ssem0, ssem1):
        cid = lax.axis_index("c")
        sub = lax.axis_index("s")

        zero16 = jnp.zeros((L,), jnp.float32)
        one16 = jnp.ones((L,), jnp.float32)

        @pl.loop(0, CHUNK)
        def _(i):
            for j in range(QUART // L):
                zero_v[i, pl.ds(j * L, L)] = zero16
            ones_v[i, pl.ds(0, L)] = one16
            cnt_v[i, pl.ds(0, L)] = zero16

        pltpu.sync_copy(nid_hbm.at[sub], nid_v)
        pltpu.sync_copy(sid_hbm.at[sub], sid_v)

        def adjust_indices(delta):
            @pl.loop(0, N_CHUNKS)
            def _(c):
                for j in range(CHUNK // L):
                    nid_v[c, pl.ds(j * L, L)] = (
                        nid_v[c, pl.ds(j * L, L)] + delta)

        @pl.loop(0, pl.cdiv(SEG_CHUNKS, NS))
        def _(z):
            t = z * NS + sub

            @pl.when(t < SEG_CHUNKS)
            def _():
                pltpu.sync_copy(zero_v, acc_sh.at[pl.ds(t * CHUNK, CHUNK)])
                pltpu.sync_copy(cnt_v, cnt_sh.at[pl.ds(t * CHUNK, CHUNK)])

        plsc.subcore_barrier()

        bufs = ((rows0, gsem0, ssem0), (rows1, gsem1, ssem1))

        def accumulate(with_counts):
            for b in range(2):
                rows, gsem, _ = bufs[b]
                pltpu.async_copy(table_hbm.at[nid_v.at[b]], rows, gsem)

            @pl.loop(0, N_CHUNKS // 2)
            def _(z):
                for b in range(2):
                    rows, gsem, ssem = bufs[b]
                    c = z * 2 + b
                    pltpu.make_async_copy(
                        table_hbm.at[nid_v.at[c]], rows, gsem).wait()
                    pltpu.async_copy(rows, acc_sh.at[sid_v.at[c]], ssem,
                                     add=True)
                    if with_counts:
                        pltpu.sync_copy(ones_v, cnt_sh.at[sid_v.at[c]],
                                        add=True)
                    pltpu.make_async_copy(rows, acc_sh.at[sid_v.at[c]],
                                          ssem).wait()
                    nxt = jnp.minimum(c + 2, N_CHUNKS - 1)

                    @pl.when(c + 2 < N_CHUNKS)
                    def _():
                        pltpu.async_copy(table_hbm.at[nid_v.at[nxt]], rows,
                                         gsem)

        def finalize(quart, rezero):
            @pl.loop(0, pl.cdiv(SEG_CHUNKS, NS))
            def _(f):
                t = f * NS + sub

                @pl.when(t < SEG_CHUNKS)
                def _():
                    base = t * CHUNK
                    pltpu.sync_copy(acc_sh.at[pl.ds(base, CHUNK)], rows0)
                    pltpu.sync_copy(cnt_sh.at[pl.ds(base, CHUNK)], cnt_v)
                    if rezero:
                        pltpu.sync_copy(zero_v,
                                        acc_sh.at[pl.ds(base, CHUNK)])

                    @pl.loop(0, CHUNK)
                    def _(i):
                        cnt = cnt_v[i, pl.ds(0, L)]
                        inv = 1.0 / jnp.maximum(cnt, 1.0)
                        for j in range(QUART // L):
                            rows0[i, pl.ds(j * L, L)] = (
                                rows0[i, pl.ds(j * L, L)] * inv)

                    pltpu.sync_copy(rows0,
                                    out_hbm.at[quart, pl.ds(base, CHUNK)])

        adjust_indices(2 * cid * N_NODES)
        accumulate(True)
        plsc.subcore_barrier()
        finalize(2 * cid, True)
        plsc.subcore_barrier()
        adjust_indices(N_NODES)
        accumulate(False)
        plsc.subcore_barrier()
        finalize(2 * cid + 1, False)

    return k(table4, nid, sid)


def kernel(embedding_table, node_ids, segment_ids):
    nid = node_ids.astype(jnp.int32)
    sid = segment_ids.astype(jnp.int32)
    pad = E_PAD - N_ENTRIES
    nid = jnp.concatenate([nid, jnp.zeros((pad,), jnp.int32)])
    sid = jnp.concatenate([sid, jnp.full((pad,), N_SEG, jnp.int32)])
    nid = nid.reshape(NS, N_CHUNKS, CHUNK)
    sid = sid.reshape(NS, N_CHUNKS, CHUNK)
    table4 = jnp.concatenate(
        [embedding_table[:, q * QUART:(q + 1) * QUART] for q in range(4)],
        axis=0)
    quarters = _sc_aggregate(table4, nid, sid)
    return jnp.concatenate([quarters[q, :N_SEG] for q in range(4)], axis=1)


# speedup vs baseline: 3.5220x; 1.0576x over previous
import functools

import jax
import jax.numpy as jnp
from jax import lax
from jax.experimental import pallas as pl
from jax.experimental.pallas import tpu as pltpu
from jax.experimental.pallas import tpu_sc as plsc

N_NODES = 10000
DIM = 256
QUART = 64
N_SEG = 10000
N_ENTRIES = 160000

NC = 2
NS = 16
L = 16

CHUNK = 128
NBUF = 4
SEG_PAD = 10112
SEG_CHUNKS = SEG_PAD // CHUNK
N_CHUNKS = 80
E_PER_SUB = N_CHUNKS * CHUNK
E_PAD = NS * E_PER_SUB

_mesh = plsc.VectorSubcoreMesh(core_axis_name="c", subcore_axis_name="s")


@jax.jit
def _sc_aggregate(table4, nid, sid):
    @functools.partial(
        pl.kernel,
        out_type=jax.ShapeDtypeStruct((2 * NC, SEG_PAD, QUART), jnp.float32),
        mesh=_mesh,
        compiler_params=pltpu.CompilerParams(use_tc_tiling_on_sc=False),
        scratch_types=[
            pltpu.VMEM((N_CHUNKS, CHUNK), jnp.int32),
            pltpu.VMEM((N_CHUNKS, CHUNK), jnp.int32),
            [pltpu.VMEM((CHUNK, QUART), jnp.float32)] * NBUF,
            pltpu.VMEM((CHUNK, QUART), jnp.float32),
            pltpu.VMEM((CHUNK, QUART), jnp.float32),
            pltpu.VMEM((CHUNK, L), jnp.float32),
            pltpu.VMEM((CHUNK, L), jnp.float32),
            pltpu.VMEM_SHARED((SEG_PAD, QUART), jnp.float32),
            pltpu.VMEM_SHARED((SEG_PAD, L), jnp.float32),
            [pltpu.SemaphoreType.DMA] * NBUF,
            [pltpu.SemaphoreType.DMA] * NBUF,
            [pltpu.SemaphoreType.DMA] * NBUF,
        ],
    )
    def k(table_hbm, nid_hbm, sid_hbm, out_hbm,
          nid_v, sid_v, rows, fin_v, zero_v, ones_v, cnt_v, acc_sh, cnt_sh,
          gsem, ssem, csem):
        cid = lax.axis_index("c")
        sub = lax.axis_index("s")

        zero16 = jnp.zeros((L,), jnp.float32)
        one16 = jnp.ones((L,), jnp.float32)

        pltpu.sync_copy(nid_hbm.at[sub], nid_v)
        pltpu.sync_copy(sid_hbm.at[sub], sid_v)

        @pl.loop(0, CHUNK)
        def _(i):
            for j in range(QUART // L):
                zero_v[i, pl.ds(j * L, L)] = zero16
            ones_v[i, pl.ds(0, L)] = one16
            cnt_v[i, pl.ds(0, L)] = zero16

        def adjust_indices(delta):
            @pl.loop(0, N_CHUNKS)
            def _(c):
                for j in range(CHUNK // L):
                    nid_v[c, pl.ds(j * L, L)] = (
                        nid_v[c, pl.ds(j * L, L)] + delta)

        def prime_gathers():
            for b in range(NBUF):
                pltpu.async_copy(table_hbm.at[nid_v.at[b]], rows[b], gsem[b])

        def accumulate(with_counts):
            @pl.loop(0, N_CHUNKS // NBUF)
            def _(z):
                for b in range(NBUF):
                    c = z * NBUF + b
                    pltpu.make_async_copy(
                        table_hbm.at[nid_v.at[c]], rows[b], gsem[b]).wait()
                    pltpu.async_copy(rows[b], acc_sh.at[sid_v.at[c]],
                                     ssem[b], add=True)
                    if with_counts:
                        @pl.when(z > 0)
                        def _():
                            pltpu.make_async_copy(
                                ones_v, cnt_sh.at[sid_v.at[c]],
                                csem[b]).wait()

                        pltpu.async_copy(ones_v, cnt_sh.at[sid_v.at[c]],
                                         csem[b], add=True)
                    pltpu.make_async_copy(rows[b], acc_sh.at[sid_v.at[c]],
                                          ssem[b]).wait()
                    nxt = jnp.minimum(c + NBUF, N_CHUNKS - 1)

                    @pl.when(c + NBUF < N_CHUNKS)
                    def _():
                        pltpu.async_copy(table_hbm.at[nid_v.at[nxt]],
                                         rows[b], gsem[b])

            if with_counts:
                for b in range(NBUF):
                    pltpu.make_async_copy(
                        ones_v, cnt_sh.at[sid_v.at[0]], csem[b]).wait()

        def finalize(quart, rezero):
            @pl.loop(0, pl.cdiv(SEG_CHUNKS, NS))
            def _(f):
                t = f * NS + sub

                @pl.when(t < SEG_CHUNKS)
                def _():
                    base = t * CHUNK
                    pltpu.sync_copy(acc_sh.at[pl.ds(base, CHUNK)], fin_v)
                    pltpu.sync_copy(cnt_sh.at[pl.ds(base, CHUNK)], cnt_v)
                    if rezero:
                        pltpu.sync_copy(zero_v,
                                        acc_sh.at[pl.ds(base, CHUNK)])

                    @pl.loop(0, CHUNK)
                    def _(i):
                        cnt = cnt_v[i, pl.ds(0, L)]
                        inv = 1.0 / jnp.maximum(cnt, 1.0)
                        for j in range(QUART // L):
                            fin_v[i, pl.ds(j * L, L)] = (
                                fin_v[i, pl.ds(j * L, L)] * inv)

                    pltpu.sync_copy(fin_v,
                                    out_hbm.at[quart, pl.ds(base, CHUNK)])

        adjust_indices(2 * cid * N_NODES)
        prime_gathers()

        @pl.loop(0, pl.cdiv(SEG_CHUNKS, NS))
        def _(z):
            t = z * NS + sub

            @pl.when(t < SEG_CHUNKS)
            def _():
                pltpu.sync_copy(zero_v, acc_sh.at[pl.ds(t * CHUNK, CHUNK)])
                pltpu.sync_copy(cnt_v, cnt_sh.at[pl.ds(t * CHUNK, CHUNK)])

        plsc.subcore_barrier()
        accumulate(True)
        adjust_indices(N_NODES)
        prime_gathers()
        plsc.subcore_barrier()
        finalize(2 * cid, True)
        plsc.subcore_barrier()
        accumulate(False)
        plsc.subcore_barrier()
        finalize(2 * cid + 1, False)

    return k(table4, nid, sid)


def kernel(embedding_table, node_ids, segment_ids):
    nid = node_ids.astype(jnp.int32)
    sid = segment_ids.astype(jnp.int32)
    pad = E_PAD - N_ENTRIES
    nid = jnp.concatenate([nid, jnp.zeros((pad,), jnp.int32)])
    sid = jnp.concatenate([sid, jnp.full((pad,), N_SEG, jnp.int32)])
    nid = nid.reshape(NS, N_CHUNKS, CHUNK)
    sid = sid.reshape(NS, N_CHUNKS, CHUNK)
    table4 = jnp.concatenate(
        [embedding_table[:, q * QUART:(q + 1) * QUART] for q in range(4)],
        axis=0)
    quarters = _sc_aggregate(table4, nid, sid)
    return jnp.concatenate([quarters[q, :N_SEG] for q in range(4)], axis=1)
